# initial kernel scaffold (unmeasured)
import jax
import jax.numpy as jnp
from jax import lax
from jax.experimental import pallas as pl
from jax.experimental.pallas import tpu as pltpu

N_DEV = 16
SQ = 512
D = 1024
HQ_LOCAL = 8
DH = 128
GROUP = 4
KV_COLS = 2 * DH
CHUNK = SQ // N_DEV
SCALE = 0.08838834764831843


def kernel(x, Wq, Wo, Wk, Wv):
    idx = lax.axis_index("i")
    wk_sl = lax.dynamic_slice_in_dim(Wk, idx * KV_COLS, KV_COLS, axis=1)
    wv_sl = lax.dynamic_slice_in_dim(Wv, idx * KV_COLS, KV_COLS, axis=1)

    def body(x_ref, wq_ref, wo_ref, wk_ref, wv_ref, out_ref,
             gath_ref, acc_ref, rs_send, rs_recv, ag_send, ag_recv):
        my = lax.axis_index("i")
        left = lax.rem(my - 1 + N_DEV, N_DEV)
        right = lax.rem(my + 1, N_DEV)

        barrier_sem = pltpu.get_barrier_semaphore()
        for nbr in (left, right):
            pl.semaphore_signal(
                barrier_sem, inc=1,
                device_id=(nbr,), device_id_type=pl.DeviceIdType.MESH,
            )
        pl.semaphore_wait(barrier_sem, 2)

        xb = x_ref[0].astype(jnp.bfloat16)
        q = jnp.dot(xb, wq_ref[...].astype(jnp.bfloat16),
                    preferred_element_type=jnp.float32)
        k = jnp.dot(xb, wk_ref[...].astype(jnp.bfloat16),
                    preferred_element_type=jnp.float32)
        v = jnp.dot(xb, wv_ref[...].astype(jnp.bfloat16),
                    preferred_element_type=jnp.float32)
        partial = jnp.zeros((SQ, D), jnp.float32)
        for h in range(HQ_LOCAL):
            kv = h // GROUP
            qh = q[:, h * DH:(h + 1) * DH].astype(jnp.bfloat16)
            kh = k[:, kv * DH:(kv + 1) * DH].astype(jnp.bfloat16)
            vh = v[:, kv * DH:(kv + 1) * DH].astype(jnp.bfloat16)
            s = lax.dot_general(qh, kh, (((1,), (1,)), ((), ())),
                                preferred_element_type=jnp.float32) * SCALE
            m = jnp.max(s, axis=1, keepdims=True)
            p = jnp.exp(s - m)
            l = jnp.sum(p, axis=1, keepdims=True)
            o = jnp.dot(p.astype(jnp.bfloat16), vh,
                        preferred_element_type=jnp.float32) / l
            partial = partial + jnp.dot(
                o.astype(jnp.bfloat16),
                wo_ref[h * DH:(h + 1) * DH, :].astype(jnp.bfloat16),
                preferred_element_type=jnp.float32)
        acc_ref[...] = partial

        gath_ref[my] = acc_ref[pl.ds(my * CHUNK, CHUNK), :]
        for s in range(N_DEV - 1):
            c = lax.rem(my - s + 2 * N_DEV, N_DEV)
            c_in = lax.rem(my - s - 1 + 2 * N_DEV, N_DEV)
            rdma = pltpu.make_async_remote_copy(
                src_ref=gath_ref.at[c],
                dst_ref=gath_ref.at[c],
                send_sem=rs_send.at[s],
                recv_sem=rs_recv.at[s],
                device_id=(right,),
                device_id_type=pl.DeviceIdType.MESH,
            )
            rdma.start()
            rdma.wait()
            gath_ref[c_in] = gath_ref[c_in] + acc_ref[pl.ds(c_in * CHUNK, CHUNK), :]

        for s in range(N_DEV - 1):
            c = lax.rem(my + 1 - s + 2 * N_DEV, N_DEV)
            rdma = pltpu.make_async_remote_copy(
                src_ref=gath_ref.at[c],
                dst_ref=gath_ref.at[c],
                send_sem=ag_send.at[s],
                recv_sem=ag_recv.at[s],
                device_id=(right,),
                device_id_type=pl.DeviceIdType.MESH,
            )
            rdma.start()
            rdma.wait()

        out_ref[0] = gath_ref[...].reshape(SQ, D)

    return pl.pallas_call(
        body,
        out_shape=jax.ShapeDtypeStruct((1, SQ, D), jnp.float32),
        in_specs=[pl.BlockSpec(memory_space=pltpu.VMEM)] * 5,
        out_specs=pl.BlockSpec(memory_space=pltpu.VMEM),
        scratch_shapes=[
            pltpu.VMEM((N_DEV, CHUNK, D), jnp.float32),
            pltpu.VMEM((SQ, D), jnp.float32),
            pltpu.SemaphoreType.DMA((N_DEV,)),
            pltpu.SemaphoreType.DMA((N_DEV,)),
            pltpu.SemaphoreType.DMA((N_DEV,)),
            pltpu.SemaphoreType.DMA((N_DEV,)),
        ],
        compiler_params=pltpu.CompilerParams(collective_id=0),
    )(x, wq_sl_dummy := Wq, Wo, wk_sl, wv_sl)


def _unused():
    pass


# baseline (device time: 118216 ns/iter reference)
import jax
import jax.numpy as jnp
from jax import lax
from jax.experimental import pallas as pl
from jax.experimental.pallas import tpu as pltpu

N_DEV = 16
SQ = 512
D = 1024
HQ_LOCAL = 8
DH = 128
GROUP = 4
KV_COLS = 2 * DH
CHUNK = SQ // N_DEV
SCALE = 0.08838834764831843


def kernel(x, Wq, Wo, Wk, Wv):
    idx = lax.axis_index("i")
    wk_sl = lax.dynamic_slice_in_dim(Wk, idx * KV_COLS, KV_COLS, axis=1)
    wv_sl = lax.dynamic_slice_in_dim(Wv, idx * KV_COLS, KV_COLS, axis=1)

    def body(x_ref, wq_ref, wo_ref, wk_ref, wv_ref, out_ref,
             gath_ref, acc_ref, rs_send, rs_recv, ag_send, ag_recv):
        my = lax.axis_index("i")
        left = lax.rem(my - 1 + N_DEV, N_DEV)
        right = lax.rem(my + 1, N_DEV)

        barrier_sem = pltpu.get_barrier_semaphore()
        for nbr in (left, right):
            pl.semaphore_signal(
                barrier_sem, inc=1,
                device_id=(nbr,), device_id_type=pl.DeviceIdType.MESH,
            )
        pl.semaphore_wait(barrier_sem, 2)

        xb = x_ref[0].astype(jnp.bfloat16)
        q = jnp.dot(xb, wq_ref[...].astype(jnp.bfloat16),
                    preferred_element_type=jnp.float32)
        k = jnp.dot(xb, wk_ref[...].astype(jnp.bfloat16),
                    preferred_element_type=jnp.float32)
        v = jnp.dot(xb, wv_ref[...].astype(jnp.bfloat16),
                    preferred_element_type=jnp.float32)
        partial = jnp.zeros((SQ, D), jnp.float32)
        for h in range(HQ_LOCAL):
            kv = h // GROUP
            qh = q[:, h * DH:(h + 1) * DH].astype(jnp.bfloat16)
            kh = k[:, kv * DH:(kv + 1) * DH].astype(jnp.bfloat16)
            vh = v[:, kv * DH:(kv + 1) * DH].astype(jnp.bfloat16)
            s = lax.dot_general(qh, kh, (((1,), (1,)), ((), ())),
                                preferred_element_type=jnp.float32) * SCALE
            m = jnp.max(s, axis=1, keepdims=True)
            p = jnp.exp(s - m)
            l = jnp.sum(p, axis=1, keepdims=True)
            o = jnp.dot(p.astype(jnp.bfloat16), vh,
                        preferred_element_type=jnp.float32) / l
            partial = partial + jnp.dot(
                o.astype(jnp.bfloat16),
                wo_ref[h * DH:(h + 1) * DH, :].astype(jnp.bfloat16),
                preferred_element_type=jnp.float32)
        acc_ref[...] = partial

        gath_ref[my] = acc_ref[pl.ds(my * CHUNK, CHUNK), :]
        for s in range(N_DEV - 1):
            c = lax.rem(my - s + 2 * N_DEV, N_DEV)
            c_in = lax.rem(my - s - 1 + 2 * N_DEV, N_DEV)
            rdma = pltpu.make_async_remote_copy(
                src_ref=gath_ref.at[c],
                dst_ref=gath_ref.at[c],
                send_sem=rs_send.at[s],
                recv_sem=rs_recv.at[s],
                device_id=(right,),
                device_id_type=pl.DeviceIdType.MESH,
            )
            rdma.start()
            rdma.wait()
            gath_ref[c_in] = gath_ref[c_in] + acc_ref[pl.ds(c_in * CHUNK, CHUNK), :]

        for s in range(N_DEV - 1):
            c = lax.rem(my + 1 - s + 2 * N_DEV, N_DEV)
            rdma = pltpu.make_async_remote_copy(
                src_ref=gath_ref.at[c],
                dst_ref=gath_ref.at[c],
                send_sem=ag_send.at[s],
                recv_sem=ag_recv.at[s],
                device_id=(right,),
                device_id_type=pl.DeviceIdType.MESH,
            )
            rdma.start()
            rdma.wait()

        out_ref[0] = gath_ref[...].reshape(SQ, D)

    return pl.pallas_call(
        body,
        out_shape=jax.ShapeDtypeStruct((1, SQ, D), jnp.float32),
        in_specs=[pl.BlockSpec(memory_space=pltpu.VMEM)] * 5,
        out_specs=pl.BlockSpec(memory_space=pltpu.VMEM),
        scratch_shapes=[
            pltpu.VMEM((N_DEV, CHUNK, D), jnp.float32),
            pltpu.VMEM((SQ, D), jnp.float32),
            pltpu.SemaphoreType.DMA((N_DEV,)),
            pltpu.SemaphoreType.DMA((N_DEV,)),
            pltpu.SemaphoreType.DMA((N_DEV,)),
            pltpu.SemaphoreType.DMA((N_DEV,)),
        ],
        compiler_params=pltpu.CompilerParams(collective_id=0),
    )(x, Wq, Wo, wk_sl, wv_sl)


# device time: 45250 ns/iter; 2.6125x vs baseline; 2.6125x over previous
import jax
import jax.numpy as jnp
from jax import lax
from jax.experimental import pallas as pl
from jax.experimental.pallas import tpu as pltpu

N_DEV = 16
SQ = 512
D = 1024
HQ_LOCAL = 8
DH = 128
GROUP = 4
KV_COLS = 2 * DH
CHUNK = SQ // N_DEV
SCALE = 0.08838834764831843


def kernel(x, Wq, Wo, Wk, Wv):
    idx = lax.axis_index("i")
    wk_sl = lax.dynamic_slice_in_dim(Wk, idx * KV_COLS, KV_COLS, axis=1)
    wv_sl = lax.dynamic_slice_in_dim(Wv, idx * KV_COLS, KV_COLS, axis=1)

    def body(x_ref, wq_ref, wo_ref, wk_ref, wv_ref, out_ref,
             send_ref, a2a_ref, gath_ref, acc_ref,
             pa_send, pa_recv, pb_send, pb_recv):
        my = lax.axis_index("i")

        barrier_sem = pltpu.get_barrier_semaphore()
        for d in range(1, N_DEV):
            tgt = lax.rem(my + d, N_DEV)
            pl.semaphore_signal(
                barrier_sem, inc=1,
                device_id=(tgt,), device_id_type=pl.DeviceIdType.MESH,
            )
        pl.semaphore_wait(barrier_sem, N_DEV - 1)

        xb = x_ref[0].astype(jnp.bfloat16)
        q = jnp.dot(xb, wq_ref[...].astype(jnp.bfloat16),
                    preferred_element_type=jnp.float32)
        k = jnp.dot(xb, wk_ref[...].astype(jnp.bfloat16),
                    preferred_element_type=jnp.float32)
        v = jnp.dot(xb, wv_ref[...].astype(jnp.bfloat16),
                    preferred_element_type=jnp.float32)
        partial = jnp.zeros((SQ, D), jnp.float32)
        for h in range(HQ_LOCAL):
            kv = h // GROUP
            qh = q[:, h * DH:(h + 1) * DH].astype(jnp.bfloat16)
            kh = k[:, kv * DH:(kv + 1) * DH].astype(jnp.bfloat16)
            vh = v[:, kv * DH:(kv + 1) * DH].astype(jnp.bfloat16)
            s = lax.dot_general(qh, kh, (((1,), (1,)), ((), ())),
                                preferred_element_type=jnp.float32) * SCALE
            m = jnp.max(s, axis=1, keepdims=True)
            p = jnp.exp(s - m)
            l = jnp.sum(p, axis=1, keepdims=True)
            o = jnp.dot(p.astype(jnp.bfloat16), vh,
                        preferred_element_type=jnp.float32) / l
            partial = partial + jnp.dot(
                o.astype(jnp.bfloat16),
                wo_ref[h * DH:(h + 1) * DH, :].astype(jnp.bfloat16),
                preferred_element_type=jnp.float32)
        acc_ref[...] = partial

        send_ref[...] = acc_ref[...].astype(jnp.bfloat16).reshape(
            N_DEV, CHUNK, D)
        a2a_ref[my] = send_ref[my]
        pa = []
        for d in range(1, N_DEV):
            tgt = lax.rem(my + d, N_DEV)
            rdma = pltpu.make_async_remote_copy(
                src_ref=send_ref.at[tgt],
                dst_ref=a2a_ref.at[my],
                send_sem=pa_send.at[tgt],
                recv_sem=pa_recv.at[my],
                device_id=(tgt,),
                device_id_type=pl.DeviceIdType.MESH,
            )
            rdma.start()
            pa.append(rdma)
        for d in range(1, N_DEV):
            src = lax.rem(my + d, N_DEV)
            pltpu.make_async_remote_copy(
                src_ref=send_ref.at[src],
                dst_ref=a2a_ref.at[src],
                send_sem=pa_send.at[src],
                recv_sem=pa_recv.at[src],
                device_id=(src,),
                device_id_type=pl.DeviceIdType.MESH,
            ).wait_recv()

        red = jnp.sum(a2a_ref[...].astype(jnp.float32), axis=0)
        gath_ref[my] = red.astype(jnp.bfloat16)
        pb = []
        for d in range(1, N_DEV):
            tgt = lax.rem(my + d, N_DEV)
            rdma = pltpu.make_async_remote_copy(
                src_ref=gath_ref.at[my],
                dst_ref=gath_ref.at[my],
                send_sem=pb_send.at[tgt],
                recv_sem=pb_recv.at[my],
                device_id=(tgt,),
                device_id_type=pl.DeviceIdType.MESH,
            )
            rdma.start()
            pb.append(rdma)
        for r in pa:
            r.wait_send()
        for d in range(1, N_DEV):
            src = lax.rem(my + d, N_DEV)
            pltpu.make_async_remote_copy(
                src_ref=gath_ref.at[src],
                dst_ref=gath_ref.at[src],
                send_sem=pb_send.at[src],
                recv_sem=pb_recv.at[src],
                device_id=(src,),
                device_id_type=pl.DeviceIdType.MESH,
            ).wait_recv()

        out_ref[0] = gath_ref[...].astype(jnp.float32).reshape(SQ, D)
        for r in pb:
            r.wait_send()

    return pl.pallas_call(
        body,
        out_shape=jax.ShapeDtypeStruct((1, SQ, D), jnp.float32),
        in_specs=[pl.BlockSpec(memory_space=pltpu.VMEM)] * 5,
        out_specs=pl.BlockSpec(memory_space=pltpu.VMEM),
        scratch_shapes=[
            pltpu.VMEM((N_DEV, CHUNK, D), jnp.bfloat16),
            pltpu.VMEM((N_DEV, CHUNK, D), jnp.bfloat16),
            pltpu.VMEM((N_DEV, CHUNK, D), jnp.bfloat16),
            pltpu.VMEM((SQ, D), jnp.float32),
            pltpu.SemaphoreType.DMA((N_DEV,)),
            pltpu.SemaphoreType.DMA((N_DEV,)),
            pltpu.SemaphoreType.DMA((N_DEV,)),
            pltpu.SemaphoreType.DMA((N_DEV,)),
        ],
        compiler_params=pltpu.CompilerParams(collective_id=0),
    )(x, Wq, Wo, wk_sl, wv_sl)
